# baseline (speedup 1.0000x reference)
import jax
import jax.numpy as jnp
from jax.experimental import pallas as pl
from jax.experimental.pallas import tpu as pltpu

_CH = 200
_NBUF = 4


def _body(x_ref, w_ref, mlp_w_ref, mlp_b_ref, adj_hbm, out_ref, xw_ref,
          b0, b1, b2, b3, s0, s1, s2, s3):
    bufs = (b0, b1, b2, b3)
    sems = (s0, s1, s2, s3)
    n = out_ref.shape[0]
    nchunks = n // _CH

    def cp(i, slot):
        return pltpu.make_async_copy(
            adj_hbm.at[pl.ds(i * _CH, _CH), :], bufs[slot], sems[slot])

    for s in range(min(_NBUF, nchunks)):
        cp(s, s).start()

    xw_ref[...] = jnp.dot(x_ref[...], w_ref[...],
                          preferred_element_type=jnp.float32)

    for i in range(nchunks):
        slot = i % _NBUF
        cp(i, slot).wait()
        a = jnp.dot(bufs[slot][...], xw_ref[...],
                    preferred_element_type=jnp.float32)
        if i + _NBUF < nchunks:
            cp(i + _NBUF, slot).start()
        a = jnp.maximum(a, 0.0)
        y = jax.lax.dot_general(a, mlp_w_ref[...],
                                dimension_numbers=(((1,), (1,)), ((), ())),
                                preferred_element_type=jnp.float32)
        y = y + mlp_b_ref[...]
        zmax = jnp.max(y, axis=1, keepdims=True)
        zmin = jnp.min(y, axis=1, keepdims=True)
        z = (y - zmin) / (zmax - zmin)
        nrm = jnp.sqrt(jnp.sum(z * z, axis=1, keepdims=True))
        out_ref[pl.ds(i * _CH, _CH), :] = z / jnp.maximum(nrm, 1e-12)


@jax.jit
def _run(x, adj, W, mlp_w, mlp_b2):
    n, d_in = x.shape
    d_hid = W.shape[1]
    d_out = mlp_w.shape[0]
    return pl.pallas_call(
        _body,
        in_specs=[
            pl.BlockSpec((n, d_in), lambda: (0, 0)),
            pl.BlockSpec((d_in, d_hid), lambda: (0, 0)),
            pl.BlockSpec((d_out, d_hid), lambda: (0, 0)),
            pl.BlockSpec((1, d_out), lambda: (0, 0)),
            pl.BlockSpec(memory_space=pltpu.MemorySpace.HBM),
        ],
        out_specs=pl.BlockSpec((n, d_out), lambda: (0, 0)),
        out_shape=jax.ShapeDtypeStruct((n, d_out), jnp.float32),
        scratch_shapes=(
            [pltpu.VMEM((n, d_hid), jnp.float32)]
            + [pltpu.VMEM((_CH, n), jnp.float32)] * _NBUF
            + [pltpu.SemaphoreType.DMA] * _NBUF
        ),
    )(x, W, mlp_w, mlp_b2, adj)


def kernel(input, adj, W, mlp_w, mlp_b):
    return _run(input, adj, W, mlp_w, mlp_b.reshape(1, -1))


# speedup vs baseline: 1.0592x; 1.0592x over previous
import jax
import jax.numpy as jnp
from jax.experimental import pallas as pl
from jax.experimental.pallas import tpu as pltpu

_CH = 400
_NBUF = 3


def _body(x_ref, w_ref, mlp_w_ref, mlp_b_ref, adj_hbm, out_hbm, xw_ref,
          b0, b1, b2, o0, o1, s0, s1, s2, t0, t1):
    bufs = (b0, b1, b2)
    sems = (s0, s1, s2)
    obufs = (o0, o1)
    osems = (t0, t1)
    n = x_ref.shape[0]
    nchunks = n // _CH

    def cp(i, slot):
        return pltpu.make_async_copy(
            adj_hbm.at[pl.ds(i * _CH, _CH), :], bufs[slot], sems[slot])

    def ocp(i, slot):
        return pltpu.make_async_copy(
            obufs[slot], out_hbm.at[pl.ds(i * _CH, _CH), :], osems[slot])

    for s in range(min(_NBUF, nchunks)):
        cp(s, s).start()

    xw_ref[...] = jnp.dot(x_ref[...], w_ref[...],
                          preferred_element_type=jnp.float32)

    for i in range(nchunks):
        slot = i % _NBUF
        oslot = i % 2
        cp(i, slot).wait()
        a = jnp.dot(bufs[slot][...], xw_ref[...],
                    preferred_element_type=jnp.float32)
        if i + _NBUF < nchunks:
            cp(i + _NBUF, slot).start()
        a = jnp.maximum(a, 0.0)
        y = jax.lax.dot_general(a, mlp_w_ref[...],
                                dimension_numbers=(((1,), (1,)), ((), ())),
                                preferred_element_type=jnp.float32)
        y = y + mlp_b_ref[...]
        zmax = jnp.max(y, axis=1, keepdims=True)
        zmin = jnp.min(y, axis=1, keepdims=True)
        z = (y - zmin) / (zmax - zmin)
        nrm = jnp.sqrt(jnp.sum(z * z, axis=1, keepdims=True))
        if i >= 2:
            ocp(i - 2, oslot).wait()
        obufs[oslot][...] = z / jnp.maximum(nrm, 1e-12)
        ocp(i, oslot).start()

    for i in range(max(nchunks - 2, 0), nchunks):
        ocp(i, i % 2).wait()


@jax.jit
def _run(x, adj, W, mlp_w, mlp_b2):
    n, d_in = x.shape
    d_hid = W.shape[1]
    d_out = mlp_w.shape[0]
    return pl.pallas_call(
        _body,
        in_specs=[
            pl.BlockSpec((n, d_in), lambda: (0, 0)),
            pl.BlockSpec((d_in, d_hid), lambda: (0, 0)),
            pl.BlockSpec((d_out, d_hid), lambda: (0, 0)),
            pl.BlockSpec((1, d_out), lambda: (0, 0)),
            pl.BlockSpec(memory_space=pltpu.MemorySpace.HBM),
        ],
        out_specs=pl.BlockSpec(memory_space=pltpu.MemorySpace.HBM),
        out_shape=jax.ShapeDtypeStruct((n, d_out), jnp.float32),
        scratch_shapes=(
            [pltpu.VMEM((n, d_hid), jnp.float32)]
            + [pltpu.VMEM((_CH, n), jnp.float32)] * _NBUF
            + [pltpu.VMEM((_CH, d_out), jnp.float32)] * 2
            + [pltpu.SemaphoreType.DMA] * (_NBUF + 2)
        ),
    )(x, W, mlp_w, mlp_b2, adj)


def kernel(input, adj, W, mlp_w, mlp_b):
    return _run(input, adj, W, mlp_w, mlp_b.reshape(1, -1))


# speedup vs baseline: 1.1020x; 1.0404x over previous
import functools

import jax
import jax.numpy as jnp
from jax.experimental import pallas as pl
from jax.experimental.pallas import tpu as pltpu


def _body(x_ref, adj_ref, w_ref, mlp_w_ref, mlp_b_ref, out_ref, xw_ref):
    @pl.when(pl.program_id(0) == 0)
    def _():
        xw_ref[...] = jnp.dot(x_ref[...], w_ref[...],
                              preferred_element_type=jnp.float32)

    a = jnp.dot(adj_ref[...], xw_ref[...], preferred_element_type=jnp.float32)
    a = jnp.maximum(a, 0.0)
    y = jax.lax.dot_general(a, mlp_w_ref[...],
                            dimension_numbers=(((1,), (1,)), ((), ())),
                            preferred_element_type=jnp.float32)
    y = y + mlp_b_ref[...]
    zmax = jnp.max(y, axis=1, keepdims=True)
    zmin = jnp.min(y, axis=1, keepdims=True)
    z = (y - zmin) / (zmax - zmin)
    nrm = jnp.sqrt(jnp.sum(z * z, axis=1, keepdims=True))
    out_ref[...] = z / jnp.maximum(nrm, 1e-12)


@functools.partial(jax.jit, static_argnames=("bm",))
def _run(x, adj, W, mlp_w, mlp_b2, bm):
    n, d_in = x.shape
    d_hid = W.shape[1]
    d_out = mlp_w.shape[0]
    return pl.pallas_call(
        _body,
        grid=(n // bm,),
        in_specs=[
            pl.BlockSpec((n, d_in), lambda i: (0, 0)),
            pl.BlockSpec((bm, n), lambda i: (i, 0)),
            pl.BlockSpec((d_in, d_hid), lambda i: (0, 0)),
            pl.BlockSpec((d_out, d_hid), lambda i: (0, 0)),
            pl.BlockSpec((1, d_out), lambda i: (0, 0)),
        ],
        out_specs=pl.BlockSpec((bm, d_out), lambda i: (i, 0)),
        out_shape=jax.ShapeDtypeStruct((n, d_out), jnp.float32),
        scratch_shapes=[pltpu.VMEM((n, d_hid), jnp.float32)],
        compiler_params=pltpu.CompilerParams(
            dimension_semantics=("arbitrary",),
        ),
    )(x, adj, W, mlp_w, mlp_b2)


def kernel(input, adj, W, mlp_w, mlp_b):
    n = input.shape[0]
    bm = next((b for b in (400, 200, 80, 40, 8, 1) if n % b == 0))
    return _run(input, adj, W, mlp_w, mlp_b.reshape(1, -1), bm)
